# baseline (speedup 1.0000x reference)
import functools

import jax
import jax.numpy as jnp
from jax import lax
from jax.experimental import pallas as pl
from jax.experimental.pallas import tpu as pltpu
from jax.experimental.pallas import tpu_sc as plsc

BATCH = 16384
EMB_DIM = 32
LANES = 16

_info = plsc.get_sparse_core_info()
NUM_CORES = _info.num_cores
NUM_SUBCORES = _info.num_subcores
NUM_WORKERS = NUM_CORES * NUM_SUBCORES
B_PER_W = BATCH // NUM_WORKERS
CHUNK = 128
N_CHUNKS = B_PER_W // CHUNK

_mesh = plsc.VectorSubcoreMesh(core_axis_name="c", subcore_axis_name="s")


@functools.partial(
    pl.kernel,
    mesh=_mesh,
    compiler_params=pltpu.CompilerParams(use_tc_tiling_on_sc=False),
    out_type=jax.ShapeDtypeStruct((BATCH, EMB_DIM), jnp.float32),
    scratch_types=[
        pltpu.VMEM((B_PER_W,), jnp.int32),
        pltpu.VMEM((B_PER_W,), jnp.int32),
        pltpu.VMEM((B_PER_W, EMB_DIM), jnp.float32),
        pltpu.VMEM((B_PER_W, EMB_DIM), jnp.float32),
        pltpu.SemaphoreType.DMA,
    ],
)
def _gmf_sc(uidx_hbm, iidx_hbm, utab_hbm, itab_hbm, out_hbm,
            uix, iix, urows, irows, sem):
  wid = lax.axis_index("s") * NUM_CORES + lax.axis_index("c")
  base = wid * B_PER_W

  pltpu.sync_copy(uidx_hbm.at[pl.ds(base, B_PER_W)], uix)
  pltpu.sync_copy(iidx_hbm.at[pl.ds(base, B_PER_W)], iix)

  copies = []
  for j in range(N_CHUNKS):
    sl = pl.ds(j * CHUNK, CHUNK)
    copies.append(pltpu.async_copy(utab_hbm.at[uix.at[sl]], urows.at[sl], sem))
    copies.append(pltpu.async_copy(itab_hbm.at[iix.at[sl]], irows.at[sl], sem))
  for c in copies:
    c.wait()

  def body(i, carry):
    lo = pl.ds(0, LANES)
    hi = pl.ds(LANES, LANES)
    urows[i, lo] = urows[i, lo] * irows[i, lo]
    urows[i, hi] = urows[i, hi] * irows[i, hi]
    return carry

  lax.fori_loop(0, B_PER_W, body, 0, unroll=8)

  pltpu.sync_copy(urows, out_hbm.at[pl.ds(base, B_PER_W)])


def kernel(user_idx, item_idx, emb_user, emb_item):
  return _gmf_sc(user_idx.astype(jnp.int32), item_idx.astype(jnp.int32),
                 emb_user, emb_item)
